# Initial kernel scaffold; baseline (speedup 1.0000x reference)
import jax
import jax.numpy as jnp
from jax.experimental import pallas as pl


def kernel(yolo_output, true_boxes, detectors_mask, matching_true_boxes, anchors):
    raise NotImplementedError("write your pallas kernel here")


# speedup vs baseline: 3.0228x; 3.0228x over previous
import functools

import jax
import jax.numpy as jnp
from jax import lax
from jax.experimental import pallas as pl
from jax.experimental.pallas import tpu as pltpu

_H = 19
_W = 19
_A = 5
_NC = 80
_CELLS = _H * _W * _A
_NB = 100


def _loss_kernel(feats_ref, tb_ref, mask_ref, mtb_ref, anc_ref, out_ref):
    f = feats_ref[...]
    inv_dim = jnp.float32(1.0 / _H)

    n = lax.broadcasted_iota(jnp.int32, (_CELLS, 1), 0)
    woff = ((n // _A) % _W).astype(jnp.float32)
    hoff = (n // (_A * _W)).astype(jnp.float32)

    x0 = f[:, 0:1]
    x1 = f[:, 1:2]
    x2 = f[:, 2:3]
    x3 = f[:, 3:4]
    x4 = f[:, 4:5]

    sx = jax.nn.sigmoid(x0)
    sy = jax.nn.sigmoid(x1)
    px = (sx + woff) * inv_dim
    py = (sy + hoff) * inv_dim
    aw = anc_ref[:, 0:1]
    ah = anc_ref[:, 1:2]
    pw = jnp.exp(x2) * aw * inv_dim
    ph = jnp.exp(x3) * ah * inv_dim

    tx = tb_ref[0:1, :]
    ty = tb_ref[1:2, :]
    tw = tb_ref[2:3, :]
    th = tb_ref[3:4, :]
    pw_half = pw * 0.5
    ph_half = ph * 0.5
    tw_half = tw * 0.5
    th_half = th * 0.5
    ix = jnp.maximum(
        jnp.minimum(px + pw_half, tx + tw_half)
        - jnp.maximum(px - pw_half, tx - tw_half), 0.0)
    iy = jnp.maximum(
        jnp.minimum(py + ph_half, ty + th_half)
        - jnp.maximum(py - ph_half, ty - th_half), 0.0)
    inter = ix * iy
    union = pw * ph + tw * th - inter
    best_iou = jnp.max(inter / union, axis=1, keepdims=True)

    m = mask_ref[...]
    conf = jax.nn.sigmoid(x4)
    obj = (best_iou > 0.6).astype(jnp.float32)
    one_m_conf = 1.0 - conf
    conf_loss = (5.0 * m) * (one_m_conf * one_m_conf) \
        + ((1.0 - obj) * (1.0 - m)) * (conf * conf)

    mt = mtb_ref[...]
    d0 = mt[:, 0:1] - sx
    d1 = mt[:, 1:2] - sy
    d2 = mt[:, 2:3] - x2
    d3 = mt[:, 3:4] - x3
    coord_loss = m * (d0 * d0 + d1 * d1 + d2 * d2 + d3 * d3)

    lane = lax.broadcasted_iota(jnp.int32, (_CELLS, _NC + 5), 1)
    is_cls = lane >= 5
    neg_inf = jnp.float32(-jnp.inf)
    cmax = jnp.max(jnp.where(is_cls, f, neg_inf), axis=1, keepdims=True)
    e = jnp.where(is_cls, jnp.exp(f - cmax), 0.0)
    s = jnp.sum(e, axis=1, keepdims=True)
    c = mt[:, 4:5].astype(jnp.int32) + 5
    hit = (lane == c)
    e_c = jnp.sum(jnp.where(hit, e, 0.0), axis=1, keepdims=True)
    cnt = jnp.sum(jnp.where(hit & is_cls, 1.0, 0.0), axis=1, keepdims=True)
    inv_s = 1.0 / s
    cls_loss = m * (jnp.sum(e * e, axis=1, keepdims=True) * (inv_s * inv_s)
                    - 2.0 * e_c * inv_s + cnt)

    total = jnp.sum(conf_loss + coord_loss + cls_loss)
    out_ref[...] = total.reshape(1, 1)


@jax.jit
def kernel(yolo_output, true_boxes, detectors_mask, matching_true_boxes, anchors):
    B = yolo_output.shape[0]
    feats = yolo_output.reshape(B, _CELLS, _NC + 5)
    tb_t = jnp.transpose(true_boxes, (0, 2, 1))
    mask = detectors_mask.reshape(B, _CELLS, 1)
    mtb = matching_true_boxes.reshape(B, _CELLS, _A)
    anc = jnp.tile(anchors, (_H * _W, 1))

    partials = pl.pallas_call(
        _loss_kernel,
        grid=(B,),
        in_specs=[
            pl.BlockSpec((None, _CELLS, _NC + 5), lambda b: (b, 0, 0)),
            pl.BlockSpec((None, 5, _NB), lambda b: (b, 0, 0)),
            pl.BlockSpec((None, _CELLS, 1), lambda b: (b, 0, 0)),
            pl.BlockSpec((None, _CELLS, _A), lambda b: (b, 0, 0)),
            pl.BlockSpec((_CELLS, 2), lambda b: (0, 0)),
        ],
        out_specs=pl.BlockSpec((None, 1, 1), lambda b: (b, 0, 0)),
        out_shape=jax.ShapeDtypeStruct((B, 1, 1), jnp.float32),
        compiler_params=pltpu.CompilerParams(
            dimension_semantics=("arbitrary",),
        ),
    )(feats, tb_t, mask, mtb, anc)
    return 0.5 * jnp.sum(partials)


# speedup vs baseline: 5.1566x; 1.7059x over previous
import jax
import jax.numpy as jnp
from jax import lax
from jax.experimental import pallas as pl
from jax.experimental.pallas import tpu as pltpu

_H = 19
_W = 19
_A = 5
_NC = 80
_CELLS = _H * _W * _A
_NB = 100


def _loss_kernel(scal_ref, cls_ref, tb_ref, geo_ref, out_ref):
    inv_dim = jnp.float32(1.0 / _H)
    sc = scal_ref[...]
    x = sc[0:1, :]
    y = sc[1:2, :]
    w = sc[2:3, :]
    h = sc[3:4, :]
    cf = sc[4:5, :]
    mbx = sc[5:6, :]
    mby = sc[6:7, :]
    mbw = sc[7:8, :]
    mbh = sc[8:9, :]
    mcls = sc[9:10, :]
    m = sc[10:11, :]

    aw = geo_ref[0:1, :]
    ah = geo_ref[1:2, :]
    woff = geo_ref[2:3, :]
    hoff = geo_ref[3:4, :]

    sx = jax.nn.sigmoid(x)
    sy = jax.nn.sigmoid(y)
    px = (sx + woff) * inv_dim
    py = (sy + hoff) * inv_dim
    pw = jnp.exp(w) * aw * inv_dim
    ph = jnp.exp(h) * ah * inv_dim

    tx = tb_ref[:, 0:1]
    ty = tb_ref[:, 1:2]
    tw = tb_ref[:, 2:3]
    th = tb_ref[:, 3:4]
    pwh = pw * 0.5
    phh = ph * 0.5
    twh = tw * 0.5
    thh = th * 0.5
    ix = jnp.maximum(
        jnp.minimum(px + pwh, tx + twh) - jnp.maximum(px - pwh, tx - twh), 0.0)
    iy = jnp.maximum(
        jnp.minimum(py + phh, ty + thh) - jnp.maximum(py - phh, ty - thh), 0.0)
    inter = ix * iy
    union = (pw * ph) + (tw * th) - inter
    best_iou = jnp.max(inter / union, axis=0, keepdims=True)

    conf = jax.nn.sigmoid(cf)
    obj = (best_iou > 0.6).astype(jnp.float32)
    one_m_conf = 1.0 - conf
    conf_loss = (5.0 * m) * (one_m_conf * one_m_conf) \
        + ((1.0 - obj) * (1.0 - m)) * (conf * conf)

    d0 = mbx - sx
    d1 = mby - sy
    d2 = mbw - w
    d3 = mbh - h
    coord_loss = m * (d0 * d0 + d1 * d1 + d2 * d2 + d3 * d3)

    cl = cls_ref[...]
    c = mcls.astype(jnp.int32)
    row = lax.broadcasted_iota(jnp.int32, (_NC, 1), 0)
    cmax = jnp.max(cl, axis=0, keepdims=True)
    e = jnp.exp(cl - cmax)
    s = jnp.sum(e, axis=0, keepdims=True)
    sum_e2 = jnp.sum(e * e, axis=0, keepdims=True)
    e_c = jnp.sum(jnp.where(row == c, e, 0.0), axis=0, keepdims=True)
    cnt = jnp.where((c >= 0) & (c < _NC), 1.0, 0.0)
    inv_s = 1.0 / s
    cls_loss = m * (sum_e2 * (inv_s * inv_s) - 2.0 * e_c * inv_s + cnt)

    total = jnp.sum(conf_loss + coord_loss + cls_loss)
    out_ref[...] = total.reshape(1, 1)


@jax.jit
def kernel(yolo_output, true_boxes, detectors_mask, matching_true_boxes, anchors):
    B = yolo_output.shape[0]
    feats = yolo_output.reshape(B, _CELLS, _NC + 5)
    cls_t = jnp.transpose(feats[:, :, 5:], (0, 2, 1))
    box_t = jnp.transpose(feats[:, :, :5], (0, 2, 1))
    mtb_t = jnp.transpose(
        matching_true_boxes.reshape(B, _CELLS, _A), (0, 2, 1))
    mask_t = detectors_mask.reshape(B, 1, _CELLS)
    scal = jnp.concatenate([box_t, mtb_t, mask_t], axis=1)

    n = jnp.arange(_CELLS, dtype=jnp.int32)
    geo = jnp.stack([
        jnp.tile(anchors[:, 0], _H * _W),
        jnp.tile(anchors[:, 1], _H * _W),
        ((n // _A) % _W).astype(jnp.float32),
        (n // (_A * _W)).astype(jnp.float32),
    ])

    partials = pl.pallas_call(
        _loss_kernel,
        grid=(B,),
        in_specs=[
            pl.BlockSpec((None, 11, _CELLS), lambda b: (b, 0, 0)),
            pl.BlockSpec((None, _NC, _CELLS), lambda b: (b, 0, 0)),
            pl.BlockSpec((None, _NB, _A), lambda b: (b, 0, 0)),
            pl.BlockSpec((4, _CELLS), lambda b: (0, 0)),
        ],
        out_specs=pl.BlockSpec((None, 1, 1), lambda b: (b, 0, 0)),
        out_shape=jax.ShapeDtypeStruct((B, 1, 1), jnp.float32),
        compiler_params=pltpu.CompilerParams(
            dimension_semantics=("arbitrary",),
        ),
    )(scal, cls_t, true_boxes, geo)
    return 0.5 * jnp.sum(partials)


# speedup vs baseline: 7.6840x; 1.4901x over previous
import jax
import jax.numpy as jnp
from jax import lax
from jax.experimental import pallas as pl
from jax.experimental.pallas import tpu as pltpu

_H = 19
_W = 19
_A = 5
_NC = 80
_CELLS = _H * _W * _A
_NB = 100


def _loss_kernel(feats_ref, scal_ref, tb_ref, geo_ref, out_ref):
    inv_dim = jnp.float32(1.0 / _H)
    ft = jnp.transpose(feats_ref[...])
    x = ft[0:1, :]
    y = ft[1:2, :]
    w = ft[2:3, :]
    h = ft[3:4, :]
    cf = ft[4:5, :]
    cl = ft[5:, :]
    sc = scal_ref[...]
    mbx = sc[0:1, :]
    mby = sc[1:2, :]
    mbw = sc[2:3, :]
    mbh = sc[3:4, :]
    mcls = sc[4:5, :]
    m = sc[5:6, :]

    aw = geo_ref[0:1, :]
    ah = geo_ref[1:2, :]
    woff = geo_ref[2:3, :]
    hoff = geo_ref[3:4, :]

    sx = jax.nn.sigmoid(x)
    sy = jax.nn.sigmoid(y)
    px = (sx + woff) * inv_dim
    py = (sy + hoff) * inv_dim
    pw = jnp.exp(w) * aw * inv_dim
    ph = jnp.exp(h) * ah * inv_dim

    tx = tb_ref[:, 0:1]
    ty = tb_ref[:, 1:2]
    tw = tb_ref[:, 2:3]
    th = tb_ref[:, 3:4]
    pwh = pw * 0.5
    phh = ph * 0.5
    twh = tw * 0.5
    thh = th * 0.5
    ix = jnp.maximum(
        jnp.minimum(px + pwh, tx + twh) - jnp.maximum(px - pwh, tx - twh), 0.0)
    iy = jnp.maximum(
        jnp.minimum(py + phh, ty + thh) - jnp.maximum(py - phh, ty - thh), 0.0)
    inter = ix * iy
    union = (pw * ph) + (tw * th) - inter
    best_iou = jnp.max(inter / union, axis=0, keepdims=True)

    conf = jax.nn.sigmoid(cf)
    obj = (best_iou > 0.6).astype(jnp.float32)
    one_m_conf = 1.0 - conf
    conf_loss = (5.0 * m) * (one_m_conf * one_m_conf) \
        + ((1.0 - obj) * (1.0 - m)) * (conf * conf)

    d0 = mbx - sx
    d1 = mby - sy
    d2 = mbw - w
    d3 = mbh - h
    coord_loss = m * (d0 * d0 + d1 * d1 + d2 * d2 + d3 * d3)

    c = mcls.astype(jnp.int32)
    row = lax.broadcasted_iota(jnp.int32, (_NC, 1), 0)
    cmax = jnp.max(cl, axis=0, keepdims=True)
    e = jnp.exp(cl - cmax)
    s = jnp.sum(e, axis=0, keepdims=True)
    sum_e2 = jnp.sum(e * e, axis=0, keepdims=True)
    e_c = jnp.sum(jnp.where(row == c, e, 0.0), axis=0, keepdims=True)
    cnt = jnp.where((c >= 0) & (c < _NC), 1.0, 0.0)
    inv_s = 1.0 / s
    cls_loss = m * (sum_e2 * (inv_s * inv_s) - 2.0 * e_c * inv_s + cnt)

    total = jnp.sum(conf_loss + coord_loss + cls_loss)
    out_ref[...] = total.reshape(1, 1)


@jax.jit
def kernel(yolo_output, true_boxes, detectors_mask, matching_true_boxes, anchors):
    B = yolo_output.shape[0]
    feats = yolo_output.reshape(B, _CELLS, _NC + 5)
    mtb_t = jnp.transpose(
        matching_true_boxes.reshape(B, _CELLS, _A), (0, 2, 1))
    mask_t = detectors_mask.reshape(B, 1, _CELLS)
    scal = jnp.concatenate([mtb_t, mask_t], axis=1)

    n = jnp.arange(_CELLS, dtype=jnp.int32)
    geo = jnp.stack([
        jnp.tile(anchors[:, 0], _H * _W),
        jnp.tile(anchors[:, 1], _H * _W),
        ((n // _A) % _W).astype(jnp.float32),
        (n // (_A * _W)).astype(jnp.float32),
    ])

    partials = pl.pallas_call(
        _loss_kernel,
        grid=(B,),
        in_specs=[
            pl.BlockSpec((None, _CELLS, _NC + 5), lambda b: (b, 0, 0)),
            pl.BlockSpec((None, 6, _CELLS), lambda b: (b, 0, 0)),
            pl.BlockSpec((None, _NB, _A), lambda b: (b, 0, 0)),
            pl.BlockSpec((4, _CELLS), lambda b: (0, 0)),
        ],
        out_specs=pl.BlockSpec((None, 1, 1), lambda b: (b, 0, 0)),
        out_shape=jax.ShapeDtypeStruct((B, 1, 1), jnp.float32),
        compiler_params=pltpu.CompilerParams(
            dimension_semantics=("arbitrary",),
        ),
    )(feats, scal, true_boxes, geo)
    return 0.5 * jnp.sum(partials)


# speedup vs baseline: 11.3170x; 1.4728x over previous
import jax
import jax.numpy as jnp
from jax import lax
from jax.experimental import pallas as pl
from jax.experimental.pallas import tpu as pltpu

_H = 19
_W = 19
_A = 5
_NC = 80
_HW = _H * _W
_CELLS = _HW * _A
_NB = 100


def _loss_kernel(feats_ref, scal_ref, tb_ref, anc_ref, out_ref):
    inv_dim = jnp.float32(1.0 / _H)
    f3 = feats_ref[...]
    f2 = f3.reshape(_HW, _A * (_NC + 5))
    ft = jnp.transpose(f2)

    k = lax.broadcasted_iota(jnp.int32, (1, _HW), 1)
    woff = (k % _W).astype(jnp.float32)
    hoff = (k // _W).astype(jnp.float32)

    tx = tb_ref[:, 0:1]
    ty = tb_ref[:, 1:2]
    tw = tb_ref[:, 2:3]
    th = tb_ref[:, 3:4]
    twh = tw * 0.5
    thh = th * 0.5
    tarea = tw * th

    row = lax.broadcasted_iota(jnp.int32, (_NC, 1), 0)

    total = jnp.zeros((), dtype=jnp.float32)
    for a in range(_A):
        base = a * (_NC + 5)
        x = ft[base + 0:base + 1, :]
        y = ft[base + 1:base + 2, :]
        w = ft[base + 2:base + 3, :]
        h = ft[base + 3:base + 4, :]
        cf = ft[base + 4:base + 5, :]
        cl = ft[base + 5:base + _NC + 5, :]

        mbx = scal_ref[a * 5 + 0:a * 5 + 1, :]
        mby = scal_ref[a * 5 + 1:a * 5 + 2, :]
        mbw = scal_ref[a * 5 + 2:a * 5 + 3, :]
        mbh = scal_ref[a * 5 + 3:a * 5 + 4, :]
        mcls = scal_ref[a * 5 + 4:a * 5 + 5, :]
        m = scal_ref[5 * _A + a:5 * _A + a + 1, :]

        aw = anc_ref[a:a + 1, 0:1]
        ah = anc_ref[a:a + 1, 1:2]

        sx = jax.nn.sigmoid(x)
        sy = jax.nn.sigmoid(y)
        px = (sx + woff) * inv_dim
        py = (sy + hoff) * inv_dim
        pw = jnp.exp(w) * (aw * inv_dim)
        ph = jnp.exp(h) * (ah * inv_dim)

        pwh = pw * 0.5
        phh = ph * 0.5
        ix = jnp.maximum(
            jnp.minimum(px + pwh, tx + twh)
            - jnp.maximum(px - pwh, tx - twh), 0.0)
        iy = jnp.maximum(
            jnp.minimum(py + phh, ty + thh)
            - jnp.maximum(py - phh, ty - thh), 0.0)
        inter = ix * iy
        union = (pw * ph) + tarea - inter
        best_iou = jnp.max(inter / union, axis=0, keepdims=True)

        conf = jax.nn.sigmoid(cf)
        obj = (best_iou > 0.6).astype(jnp.float32)
        one_m_conf = 1.0 - conf
        conf_loss = (5.0 * m) * (one_m_conf * one_m_conf) \
            + ((1.0 - obj) * (1.0 - m)) * (conf * conf)

        d0 = mbx - sx
        d1 = mby - sy
        d2 = mbw - w
        d3 = mbh - h
        coord_loss = m * (d0 * d0 + d1 * d1 + d2 * d2 + d3 * d3)

        c = mcls.astype(jnp.int32)
        cmax = jnp.max(cl, axis=0, keepdims=True)
        e = jnp.exp(cl - cmax)
        s = jnp.sum(e, axis=0, keepdims=True)
        sum_e2 = jnp.sum(e * e, axis=0, keepdims=True)
        e_c = jnp.sum(jnp.where(row == c, e, 0.0), axis=0, keepdims=True)
        cnt = jnp.where((c >= 0) & (c < _NC), 1.0, 0.0)
        inv_s = 1.0 / s
        cls_loss = m * (sum_e2 * (inv_s * inv_s) - 2.0 * e_c * inv_s + cnt)

        total = total + jnp.sum(conf_loss + coord_loss + cls_loss)

    out_ref[...] = total.reshape(1, 1)


@jax.jit
def kernel(yolo_output, true_boxes, detectors_mask, matching_true_boxes, anchors):
    B = yolo_output.shape[0]
    mtb_t = jnp.transpose(
        matching_true_boxes.reshape(B, _HW, _A, _A), (0, 2, 3, 1)
    ).reshape(B, _A * _A, _HW)
    mask_t = jnp.transpose(
        detectors_mask.reshape(B, _HW, _A), (0, 2, 1))
    scal = jnp.concatenate([mtb_t, mask_t], axis=1)

    partials = pl.pallas_call(
        _loss_kernel,
        grid=(B,),
        in_specs=[
            pl.BlockSpec((None, _H, _W, _A * (_NC + 5)), lambda b: (b, 0, 0, 0)),
            pl.BlockSpec((None, 6 * _A, _HW), lambda b: (b, 0, 0)),
            pl.BlockSpec((None, _NB, _A), lambda b: (b, 0, 0)),
            pl.BlockSpec((_A, 2), lambda b: (0, 0)),
        ],
        out_specs=pl.BlockSpec((None, 1, 1), lambda b: (b, 0, 0)),
        out_shape=jax.ShapeDtypeStruct((B, 1, 1), jnp.float32),
        compiler_params=pltpu.CompilerParams(
            dimension_semantics=("arbitrary",),
        ),
    )(yolo_output, scal, true_boxes, anchors)
    return 0.5 * jnp.sum(partials)


# speedup vs baseline: 11.5494x; 1.0205x over previous
import jax
import jax.numpy as jnp
from jax import lax
from jax.experimental import pallas as pl
from jax.experimental.pallas import tpu as pltpu

_H = 19
_W = 19
_A = 5
_NC = 80
_HW = _H * _W
_CELLS = _HW * _A
_NB = 100


def _loss_kernel(feats_ref, scal_ref, tb_ref, anc_ref, out_ref):
    inv_dim = jnp.float32(1.0 / _H)
    f3 = feats_ref[...]
    f2 = f3.reshape(_HW, _A * (_NC + 5))
    ft = jnp.transpose(f2)

    k = lax.broadcasted_iota(jnp.int32, (1, _HW), 1)
    woff = (k % _W).astype(jnp.float32)
    hoff = (k // _W).astype(jnp.float32)

    tx = tb_ref[:, 0:1]
    ty = tb_ref[:, 1:2]
    tw = tb_ref[:, 2:3]
    th = tb_ref[:, 3:4]
    twh = tw * 0.5
    thh = th * 0.5
    tarea6 = (tw * th) * 0.6

    row = lax.broadcasted_iota(jnp.int32, (_NC, 1), 0)

    total = jnp.zeros((), dtype=jnp.float32)
    for a in range(_A):
        base = a * (_NC + 5)
        x = ft[base + 0:base + 1, :]
        y = ft[base + 1:base + 2, :]
        w = ft[base + 2:base + 3, :]
        h = ft[base + 3:base + 4, :]
        cf = ft[base + 4:base + 5, :]
        cl = ft[base + 5:base + _NC + 5, :]

        mbx = scal_ref[a * 5 + 0:a * 5 + 1, :]
        mby = scal_ref[a * 5 + 1:a * 5 + 2, :]
        mbw = scal_ref[a * 5 + 2:a * 5 + 3, :]
        mbh = scal_ref[a * 5 + 3:a * 5 + 4, :]
        mcls = scal_ref[a * 5 + 4:a * 5 + 5, :]
        m = scal_ref[5 * _A + a:5 * _A + a + 1, :]

        aw = anc_ref[a:a + 1, 0:1]
        ah = anc_ref[a:a + 1, 1:2]

        sx = jax.nn.sigmoid(x)
        sy = jax.nn.sigmoid(y)
        px = (sx + woff) * inv_dim
        py = (sy + hoff) * inv_dim
        pw = jnp.exp(w) * (aw * inv_dim)
        ph = jnp.exp(h) * (ah * inv_dim)

        pwh = pw * 0.5
        phh = ph * 0.5
        ix = jnp.maximum(
            jnp.minimum(px + pwh, tx + twh)
            - jnp.maximum(px - pwh, tx - twh), 0.0)
        iy = jnp.maximum(
            jnp.minimum(py + phh, ty + thh)
            - jnp.maximum(py - phh, ty - thh), 0.0)
        inter = ix * iy
        score = 1.6 * inter - ((0.6 * (pw * ph)) + tarea6)
        best = jnp.max(score, axis=0, keepdims=True)

        conf = jax.nn.sigmoid(cf)
        obj = (best > 0.0).astype(jnp.float32)
        one_m_conf = 1.0 - conf
        conf_loss = (5.0 * m) * (one_m_conf * one_m_conf) \
            + ((1.0 - obj) * (1.0 - m)) * (conf * conf)

        d0 = mbx - sx
        d1 = mby - sy
        d2 = mbw - w
        d3 = mbh - h
        coord_loss = m * (d0 * d0 + d1 * d1 + d2 * d2 + d3 * d3)

        c = mcls.astype(jnp.int32)
        cmax = jnp.max(cl, axis=0, keepdims=True)
        e = jnp.exp(cl - cmax)
        s = jnp.sum(e, axis=0, keepdims=True)
        sum_e2 = jnp.sum(e * e, axis=0, keepdims=True)
        e_c = jnp.sum(jnp.where(row == c, e, 0.0), axis=0, keepdims=True)
        cnt = jnp.where((c >= 0) & (c < _NC), 1.0, 0.0)
        inv_s = 1.0 / s
        cls_loss = m * (sum_e2 * (inv_s * inv_s) - 2.0 * e_c * inv_s + cnt)

        total = total + jnp.sum(conf_loss + coord_loss + cls_loss)

    out_ref[...] = total.reshape(1, 1)


@jax.jit
def kernel(yolo_output, true_boxes, detectors_mask, matching_true_boxes, anchors):
    B = yolo_output.shape[0]
    mtb_t = jnp.transpose(
        matching_true_boxes.reshape(B, _HW, _A, _A), (0, 2, 3, 1)
    ).reshape(B, _A * _A, _HW)
    mask_t = jnp.transpose(
        detectors_mask.reshape(B, _HW, _A), (0, 2, 1))
    scal = jnp.concatenate([mtb_t, mask_t], axis=1)

    partials = pl.pallas_call(
        _loss_kernel,
        grid=(B,),
        in_specs=[
            pl.BlockSpec((None, _H, _W, _A * (_NC + 5)), lambda b: (b, 0, 0, 0)),
            pl.BlockSpec((None, 6 * _A, _HW), lambda b: (b, 0, 0)),
            pl.BlockSpec((None, _NB, _A), lambda b: (b, 0, 0)),
            pl.BlockSpec((_A, 2), lambda b: (0, 0)),
        ],
        out_specs=pl.BlockSpec((None, 1, 1), lambda b: (b, 0, 0)),
        out_shape=jax.ShapeDtypeStruct((B, 1, 1), jnp.float32),
        compiler_params=pltpu.CompilerParams(
            dimension_semantics=("arbitrary",),
        ),
    )(yolo_output, scal, true_boxes, anchors)
    return 0.5 * jnp.sum(partials)


# speedup vs baseline: 12.0655x; 1.0447x over previous
import jax
import jax.numpy as jnp
from jax import lax
from jax.experimental import pallas as pl
from jax.experimental.pallas import tpu as pltpu

_H = 19
_W = 19
_A = 5
_NC = 80
_HW = _H * _W
_CELLS = _HW * _A
_NB = 100


def _loss_kernel(feats_ref, scal_ref, tb_ref, anc_ref, out_ref):
    inv_dim = jnp.float32(1.0 / _H)
    f3 = feats_ref[...]
    f2 = f3.reshape(_HW, _A * (_NC + 5))
    ft = jnp.transpose(f2)

    k = lax.broadcasted_iota(jnp.int32, (1, _HW), 1)
    woff = (k % _W).astype(jnp.float32)
    hoff = (k // _W).astype(jnp.float32)

    tx = tb_ref[:, 0:1]
    ty = tb_ref[:, 1:2]
    tw = tb_ref[:, 2:3]
    th = tb_ref[:, 3:4]
    twh = tw * 0.5
    thh = th * 0.5
    tminx = tx - twh
    tmaxx = tx + twh
    tminy = ty - thh
    tmaxy = ty + thh
    tarea6 = (tw * th) * 0.6

    row = lax.broadcasted_iota(jnp.int32, (_NC, 1), 0)
    ones_row = jnp.ones((1, _NC), dtype=jnp.float32)

    total = jnp.zeros((), dtype=jnp.float32)
    for a in range(_A):
        base = a * (_NC + 5)
        x = ft[base + 0:base + 1, :]
        y = ft[base + 1:base + 2, :]
        w = ft[base + 2:base + 3, :]
        h = ft[base + 3:base + 4, :]
        cf = ft[base + 4:base + 5, :]
        cl = ft[base + 5:base + _NC + 5, :]

        mbx = scal_ref[a * 5 + 0:a * 5 + 1, :]
        mby = scal_ref[a * 5 + 1:a * 5 + 2, :]
        mbw = scal_ref[a * 5 + 2:a * 5 + 3, :]
        mbh = scal_ref[a * 5 + 3:a * 5 + 4, :]
        mcls = scal_ref[a * 5 + 4:a * 5 + 5, :]
        m = scal_ref[5 * _A + a:5 * _A + a + 1, :]

        aw = anc_ref[a:a + 1, 0:1]
        ah = anc_ref[a:a + 1, 1:2]

        sx = jax.nn.sigmoid(x)
        sy = jax.nn.sigmoid(y)
        px = (sx + woff) * inv_dim
        py = (sy + hoff) * inv_dim
        pw = jnp.exp(w) * (aw * inv_dim)
        ph = jnp.exp(h) * (ah * inv_dim)

        pwh = pw * 0.5
        phh = ph * 0.5
        ix = jnp.maximum(
            jnp.minimum(px + pwh, tmaxx) - jnp.maximum(px - pwh, tminx), 0.0)
        iy = jnp.maximum(
            jnp.minimum(py + phh, tmaxy) - jnp.maximum(py - phh, tminy), 0.0)
        inter = ix * iy
        score = 1.6 * inter - ((0.6 * (pw * ph)) + tarea6)
        best = jnp.max(score, axis=0, keepdims=True)

        conf = jax.nn.sigmoid(cf)
        obj = (best > 0.0).astype(jnp.float32)
        one_m_conf = 1.0 - conf
        conf_loss = (5.0 * m) * (one_m_conf * one_m_conf) \
            + ((1.0 - obj) * (1.0 - m)) * (conf * conf)

        d0 = mbx - sx
        d1 = mby - sy
        d2 = mbw - w
        d3 = mbh - h
        coord_loss = m * (d0 * d0 + d1 * d1 + d2 * d2 + d3 * d3)

        c = mcls.astype(jnp.int32)
        cmax = jnp.max(cl, axis=0, keepdims=True)
        e = jnp.exp(cl - cmax)
        dnums = (((1,), (0,)), ((), ()))
        s = lax.dot_general(ones_row, e, dnums,
                            preferred_element_type=jnp.float32)
        sum_e2 = lax.dot_general(ones_row, e * e, dnums,
                                 preferred_element_type=jnp.float32)
        e_c = lax.dot_general(ones_row, jnp.where(row == c, e, 0.0), dnums,
                              preferred_element_type=jnp.float32)
        cnt = jnp.where((c >= 0) & (c < _NC), 1.0, 0.0)
        inv_s = 1.0 / s
        cls_loss = m * (sum_e2 * (inv_s * inv_s) - 2.0 * e_c * inv_s + cnt)

        total = total + jnp.sum(conf_loss + coord_loss + cls_loss)

    out_ref[...] = total.reshape(1, 1)


@jax.jit
def kernel(yolo_output, true_boxes, detectors_mask, matching_true_boxes, anchors):
    B = yolo_output.shape[0]
    mtb_t = jnp.transpose(
        matching_true_boxes.reshape(B, _HW, _A, _A), (0, 2, 3, 1)
    ).reshape(B, _A * _A, _HW)
    mask_t = jnp.transpose(
        detectors_mask.reshape(B, _HW, _A), (0, 2, 1))
    scal = jnp.concatenate([mtb_t, mask_t], axis=1)

    partials = pl.pallas_call(
        _loss_kernel,
        grid=(B,),
        in_specs=[
            pl.BlockSpec((None, _H, _W, _A * (_NC + 5)), lambda b: (b, 0, 0, 0)),
            pl.BlockSpec((None, 6 * _A, _HW), lambda b: (b, 0, 0)),
            pl.BlockSpec((None, _NB, _A), lambda b: (b, 0, 0)),
            pl.BlockSpec((_A, 2), lambda b: (0, 0)),
        ],
        out_specs=pl.BlockSpec((None, 1, 1), lambda b: (b, 0, 0)),
        out_shape=jax.ShapeDtypeStruct((B, 1, 1), jnp.float32),
        compiler_params=pltpu.CompilerParams(
            dimension_semantics=("arbitrary",),
        ),
    )(yolo_output, scal, true_boxes, anchors)
    return 0.5 * jnp.sum(partials)
